# baseline (speedup 1.0000x reference)
import jax
import jax.numpy as jnp
from jax.experimental import pallas as pl
from jax.experimental.pallas import tpu as pltpu


def _split_hi_lo(v):
    hi = v.astype(jnp.bfloat16)
    lo = (v - hi.astype(jnp.float32)).astype(jnp.bfloat16)
    return hi, lo


def _mwc_kernel(
    eo_ref, ec_ref, c_ref, idx_ref, eps_ref, out_ref,
    ph_scr, plo_scr, dh_scr, dlo_scr,
):
    ir = pl.program_id(0)
    n_units = eo_ref.shape[1]
    bb = out_ref.shape[0]
    br = out_ref.shape[1]

    def _build_s():
        idx = idx_ref[...]
        u_iota = jax.lax.broadcasted_iota(jnp.int32, (n_units, br), 0)
        s = jnp.zeros((n_units, br), jnp.float32)
        for k in range(idx_ref.shape[0]):
            s = s + jnp.where(u_iota == idx[k : k + 1, :], 1.0, 0.0)
        return s.astype(jnp.bfloat16)

    def _epilogue(x, er, sd):
        L = jnp.exp(-er)
        p_min = 1.0 / (1.0 + L)
        p_c = 1.0 / (1.0 + L * jnp.exp(x))
        p_max = 1.0 / (1.0 + L * jnp.exp(sd))
        denom = p_max - p_min
        norm = (p_c - p_min) / (denom + 1e-8)
        norm = jnp.where(denom > 1e-6, norm, 0.0)
        out_ref[...] = jnp.clip(norm, 0.0, 1.0)

    def _main(sb):
        xf = jnp.dot(
            ph_scr[...], sb, preferred_element_type=jnp.float32
        ) + jnp.dot(plo_scr[...], sb, preferred_element_type=jnp.float32)
        sd = jnp.dot(
            dh_scr[...], sb, preferred_element_type=jnp.float32
        ) + jnp.dot(dlo_scr[...], sb, preferred_element_type=jnp.float32)
        _epilogue(xf[:bb, :], xf[bb : bb + 1, :], sd)

    @pl.when(ir == 0)
    def _():
        sb = _build_s()
        c = c_ref[...]
        xf_acc = None
        sd_acc = None
        er_acc = None
        for lo, w in ((0, 512), (512, n_units - 512)):
            eo = eo_ref[:, pl.ds(lo, w)]
            ec = ec_ref[:, pl.ds(lo, w)]
            p = jnp.log1p(c * jnp.exp(-ec)) - jnp.log1p(c * jnp.exp(-eo))
            ph, plo = _split_hi_lo(p)
            eh, elo = _split_hi_lo(eps_ref[:, pl.ds(lo, w)])
            ph_scr[0:bb, pl.ds(lo, w)] = ph
            ph_scr[bb : bb + 1, pl.ds(lo, w)] = eh
            plo_scr[0:bb, pl.ds(lo, w)] = plo
            plo_scr[bb : bb + 1, pl.ds(lo, w)] = elo
            dh, dlo = _split_hi_lo(eo - ec)
            dh_scr[:, pl.ds(lo, w)] = dh
            dlo_scr[:, pl.ds(lo, w)] = dlo
            sb_h = sb[lo : lo + w, :]
            f32 = jnp.float32
            xf_h = jnp.dot(ph, sb_h, preferred_element_type=f32) + jnp.dot(
                plo, sb_h, preferred_element_type=f32
            )
            sd_h = jnp.dot(dh, sb_h, preferred_element_type=f32) + jnp.dot(
                dlo, sb_h, preferred_element_type=f32
            )
            er_h = jnp.dot(eh, sb_h, preferred_element_type=f32) + jnp.dot(
                elo, sb_h, preferred_element_type=f32
            )
            xf_acc = xf_h if xf_acc is None else xf_acc + xf_h
            sd_acc = sd_h if sd_acc is None else sd_acc + sd_h
            er_acc = er_h if er_acc is None else er_acc + er_h
        _epilogue(xf_acc, er_acc, sd_acc)

    @pl.when(ir != 0)
    def _():
        _main(_build_s())


@jax.jit
def kernel(energies, concentrations, receptor_indices, epsilon_units):
    b, u, _ = energies.shape
    r, k = receptor_indices.shape
    br = 512
    nr = r // br

    one = jnp.float32(1.0)
    eo = energies[:, :, 0] * one
    ec = energies[:, :, 1] * one
    c2 = concentrations.reshape(b, 1)
    idxt = receptor_indices.T
    eps2 = epsilon_units.reshape(1, u)

    return pl.pallas_call(
        _mwc_kernel,
        grid=(nr,),
        in_specs=[
            pl.BlockSpec((b, u), lambda ir: (0, 0)),
            pl.BlockSpec((b, u), lambda ir: (0, 0)),
            pl.BlockSpec((b, 1), lambda ir: (0, 0)),
            pl.BlockSpec((k, br), lambda ir: (0, ir)),
            pl.BlockSpec((1, u), lambda ir: (0, 0)),
        ],
        out_specs=pl.BlockSpec((b, br), lambda ir: (0, ir)),
        out_shape=jax.ShapeDtypeStruct((b, r), jnp.float32),
        scratch_shapes=[
            pltpu.VMEM((b + 8, u), jnp.bfloat16),
            pltpu.VMEM((b + 8, u), jnp.bfloat16),
            pltpu.VMEM((b, u), jnp.bfloat16),
            pltpu.VMEM((b, u), jnp.bfloat16),
        ],
    )(eo, ec, c2, idxt, eps2)
